# baseline (speedup 1.0000x reference)
import functools

import jax
import jax.numpy as jnp
from jax import lax
from jax.experimental import pallas as pl
from jax.experimental.pallas import tpu as pltpu
from jax.experimental.pallas import tpu_sc as plsc

_BATCH = 16384
_L = 16
_NC = 2
_NS = 16
_NW = _NC * _NS
_B_PER_W = _BATCH // _NW
_CHUNK = 128
_NCHUNK = _B_PER_W // _CHUNK

_mesh = plsc.VectorSubcoreMesh(core_axis_name="c", subcore_axis_name="s")


@functools.partial(
    pl.kernel,
    mesh=_mesh,
    out_type=jax.ShapeDtypeStruct((_NC, _L), jnp.float32),
    scratch_types=[
        pltpu.VMEM((_B_PER_W,), jnp.int32),
        pltpu.VMEM((_B_PER_W,), jnp.float32),
        pltpu.VMEM((_B_PER_W,), jnp.float32),
        pltpu.VMEM((_B_PER_W,), jnp.float32),
        pltpu.VMEM((_L,), jnp.float32),
        pltpu.VMEM((_L,), jnp.float32),
        pltpu.VMEM((_NS, _L), jnp.float32),
        pltpu.VMEM_SHARED((_NS, _L), jnp.float32),
        pltpu.SemaphoreType.DMA,
        pltpu.SemaphoreType.DMA,
        pltpu.SemaphoreType.DMA,
        pltpu.SemaphoreType.DMA,
    ],
)
def _sc_gather_sum(idx_hbm, a_hbm, b_hbm, c_hbm, m_hbm, out_hbm,
                   idx_v, av, bv, cv, mv, pv, gv, shared,
                   sem0, sem1, sem2, sem3):
    cid = lax.axis_index("c")
    sid = lax.axis_index("s")
    wid = sid * _NC + cid
    sems = (sem0, sem1, sem2, sem3)

    idx_copies = [
        pltpu.async_copy(
            idx_hbm.at[pl.ds(wid * _B_PER_W + j * _CHUNK, _CHUNK)],
            idx_v.at[pl.ds(j * _CHUNK, _CHUNK)], sems[j])
        for j in range(_NCHUNK)
    ]

    copies = []
    for j in range(_NCHUNK):
        s = pl.ds(j * _CHUNK, _CHUNK)
        idx_copies[j].wait()
        copies.append((pltpu.async_copy(a_hbm.at[idx_v.at[s]], av.at[s], sems[j]),
                       pltpu.async_copy(b_hbm.at[idx_v.at[s]], bv.at[s], sems[j]),
                       pltpu.async_copy(c_hbm.at[idx_v.at[s]], cv.at[s], sems[j])))
    pltpu.sync_copy(m_hbm, mv)

    acc_ac = jnp.zeros((_L,), jnp.float32)
    acc_b = jnp.zeros((_L,), jnp.float32)
    for j in range(_NCHUNK):
        for cp in copies[j]:
            cp.wait()

        def body(i, accs):
            a_ac, a_b = accs
            s = pl.ds(j * _CHUNK + i * _L, _L)
            return a_ac + av[s] + cv[s], a_b + bv[s]

        acc_ac, acc_b = lax.fori_loop(0, _CHUNK // _L, body,
                                      (acc_ac, acc_b), unroll=2)

    pv[...] = acc_ac * mv[...] + acc_b
    pltpu.sync_copy(pv, shared.at[sid])
    plsc.subcore_barrier()

    @pl.when(sid == 0)
    def _():
        pltpu.sync_copy(shared, gv)
        total = gv[0, :]
        for i in range(1, _NS):
            total = total + gv[i, :]
        pv[...] = total
        pltpu.sync_copy(pv, out_hbm.at[cid])


def kernel(c_t_indices, M_prev, A, B, C):
    m16 = jnp.full((_L,), M_prev, jnp.float32)
    partials = _sc_gather_sum(c_t_indices.astype(jnp.int32), A, B, C, m16)
    return jnp.sum(partials)


# speedup vs baseline: 1.0044x; 1.0044x over previous
import functools

import jax
import jax.numpy as jnp
from jax import lax
from jax.experimental import pallas as pl
from jax.experimental.pallas import tpu as pltpu
from jax.experimental.pallas import tpu_sc as plsc

_BATCH = 16384
_L = 16
_NC = 2
_NS = 16
_NW = _NC * _NS
_B_PER_W = _BATCH // _NW
_CHUNK = 128
_NCHUNK = _B_PER_W // _CHUNK

_mesh = plsc.VectorSubcoreMesh(core_axis_name="c", subcore_axis_name="s")


@functools.partial(
    pl.kernel,
    mesh=_mesh,
    out_type=jax.ShapeDtypeStruct((2 * _NW, _L), jnp.float32),
    scratch_types=[
        pltpu.VMEM((_B_PER_W,), jnp.int32),
        pltpu.VMEM((_B_PER_W,), jnp.float32),
        pltpu.VMEM((_B_PER_W,), jnp.float32),
        pltpu.VMEM((_B_PER_W,), jnp.float32),
        pltpu.VMEM((_L,), jnp.float32),
        pltpu.VMEM((_L,), jnp.float32),
        pltpu.VMEM((_L,), jnp.float32),
        pltpu.SemaphoreType.DMA,
        pltpu.SemaphoreType.DMA,
        pltpu.SemaphoreType.DMA,
        pltpu.SemaphoreType.DMA,
    ],
)
def _sc_gather_sum(idx_hbm, a_hbm, b_hbm, c_hbm, m_hbm, out_hbm,
                   idx_v, av, bv, cv, mv, pv, qv,
                   sem0, sem1, sem2, sem3):
    cid = lax.axis_index("c")
    sid = lax.axis_index("s")
    wid = sid * _NC + cid
    sems = (sem0, sem1, sem2, sem3)

    idx_copies = [
        pltpu.async_copy(
            idx_hbm.at[pl.ds(wid * _B_PER_W + j * _CHUNK, _CHUNK)],
            idx_v.at[pl.ds(j * _CHUNK, _CHUNK)], sems[j])
        for j in range(_NCHUNK)
    ]

    copies = []
    for j in range(_NCHUNK):
        s = pl.ds(j * _CHUNK, _CHUNK)
        idx_copies[j].wait()
        copies.append((pltpu.async_copy(a_hbm.at[idx_v.at[s]], av.at[s], sems[j]),
                       pltpu.async_copy(b_hbm.at[idx_v.at[s]], bv.at[s], sems[j]),
                       pltpu.async_copy(c_hbm.at[idx_v.at[s]], cv.at[s], sems[j])))
    pltpu.sync_copy(m_hbm, mv)

    half_copies = []
    for h in range(2):
        acc_ac = jnp.zeros((_L,), jnp.float32)
        acc_b = jnp.zeros((_L,), jnp.float32)
        for j in range(h * _NCHUNK // 2, (h + 1) * _NCHUNK // 2):
            for cp in copies[j]:
                cp.wait()

            def body(i, accs):
                a_ac, a_b = accs
                s = pl.ds(j * _CHUNK + i * _L, _L)
                return a_ac + av[s] + cv[s], a_b + bv[s]

            acc_ac, acc_b = lax.fori_loop(0, _CHUNK // _L, body,
                                          (acc_ac, acc_b), unroll=2)
        p = (pv, qv)[h]
        p[...] = acc_ac * mv[...] + acc_b
        half_copies.append(
            pltpu.async_copy(p, out_hbm.at[h * _NW + wid], sems[h]))
    for cp in half_copies:
        cp.wait()


def kernel(c_t_indices, M_prev, A, B, C):
    m16 = jnp.full((_L,), M_prev, jnp.float32)
    partials = _sc_gather_sum(c_t_indices.astype(jnp.int32), A, B, C, m16)
    return jnp.sum(partials)


# speedup vs baseline: 1.0051x; 1.0007x over previous
import functools

import jax
import jax.numpy as jnp
from jax import lax
from jax.experimental import pallas as pl
from jax.experimental.pallas import tpu as pltpu
from jax.experimental.pallas import tpu_sc as plsc

_BATCH = 16384
_L = 16
_NC = 2
_NS = 16
_NW = _NC * _NS
_B_PER_W = _BATCH // _NW
_CHUNK = 128
_NCHUNK = _B_PER_W // _CHUNK

_mesh = plsc.VectorSubcoreMesh(core_axis_name="c", subcore_axis_name="s")


@functools.partial(
    pl.kernel,
    mesh=_mesh,
    out_type=jax.ShapeDtypeStruct((_NW, _L), jnp.float32),
    scratch_types=[
        pltpu.VMEM((_B_PER_W,), jnp.int32),
        pltpu.VMEM((_B_PER_W,), jnp.float32),
        pltpu.VMEM((_B_PER_W,), jnp.float32),
        pltpu.VMEM((_B_PER_W,), jnp.float32),
        pltpu.VMEM((_L,), jnp.float32),
        pltpu.VMEM((_L,), jnp.float32),
        pltpu.SemaphoreType.DMA,
        pltpu.SemaphoreType.DMA,
        pltpu.SemaphoreType.DMA,
        pltpu.SemaphoreType.DMA,
    ],
)
def _sc_gather_sum(idx_hbm, a_hbm, b_hbm, c_hbm, m_hbm, out_hbm,
                   idx_v, av, bv, cv, mv, pv,
                   sem0, sem1, sem2, sem3):
    cid = lax.axis_index("c")
    sid = lax.axis_index("s")
    wid = sid * _NC + cid
    sems = (sem0, sem1, sem2, sem3)

    idx_copies = [
        pltpu.async_copy(
            idx_hbm.at[pl.ds(wid * _B_PER_W + j * _CHUNK, _CHUNK)],
            idx_v.at[pl.ds(j * _CHUNK, _CHUNK)], sems[j])
        for j in range(_NCHUNK)
    ]

    copies = []
    for j in range(_NCHUNK):
        s = pl.ds(j * _CHUNK, _CHUNK)
        idx_copies[j].wait()
        copies.append((pltpu.async_copy(a_hbm.at[idx_v.at[s]], av.at[s], sems[j]),
                       pltpu.async_copy(b_hbm.at[idx_v.at[s]], bv.at[s], sems[j]),
                       pltpu.async_copy(c_hbm.at[idx_v.at[s]], cv.at[s], sems[j])))
    pltpu.sync_copy(m_hbm, mv)

    acc_ac = jnp.zeros((_L,), jnp.float32)
    acc_b = jnp.zeros((_L,), jnp.float32)
    for j in range(_NCHUNK):
        for cp in copies[j]:
            cp.wait()

        def body(i, accs):
            a_ac, a_b = accs
            s = pl.ds(j * _CHUNK + i * _L, _L)
            return a_ac + av[s] + cv[s], a_b + bv[s]

        acc_ac, acc_b = lax.fori_loop(0, _CHUNK // _L, body,
                                      (acc_ac, acc_b), unroll=2)

    pv[...] = acc_ac * mv[...] + acc_b
    pltpu.sync_copy(pv, out_hbm.at[wid])


def kernel(c_t_indices, M_prev, A, B, C):
    m16 = jnp.full((_L,), M_prev, jnp.float32)
    partials = _sc_gather_sum(c_t_indices.astype(jnp.int32), A, B, C, m16)
    return jnp.sum(partials)


# speedup vs baseline: 1.0343x; 1.0291x over previous
import functools

import jax
import jax.numpy as jnp
from jax import lax
from jax.experimental import pallas as pl
from jax.experimental.pallas import tpu as pltpu
from jax.experimental.pallas import tpu_sc as plsc

_BATCH = 16384
_L = 16
_NC = 2
_NS = 16
_NW = _NC * _NS
_B_PER_W = _BATCH // _NW
_CHUNK = 128
_NCHUNK = _B_PER_W // _CHUNK

_mesh = plsc.VectorSubcoreMesh(core_axis_name="c", subcore_axis_name="s")


@functools.partial(
    pl.kernel,
    mesh=_mesh,
    out_type=jax.ShapeDtypeStruct((_NW, _L), jnp.float32),
    scratch_types=[
        pltpu.VMEM((_B_PER_W,), jnp.int32),
        pltpu.VMEM((_B_PER_W,), jnp.float32),
        pltpu.VMEM((_B_PER_W,), jnp.float32),
        pltpu.VMEM((_B_PER_W,), jnp.float32),
        pltpu.VMEM((_L,), jnp.float32),
        pltpu.VMEM((_L,), jnp.float32),
        pltpu.SemaphoreType.DMA,
        pltpu.SemaphoreType.DMA,
        pltpu.SemaphoreType.DMA,
        pltpu.SemaphoreType.DMA,
    ],
)
def _sc_gather_sum(idx_hbm, a_hbm, b_hbm, c_hbm, m_hbm, out_hbm,
                   idx_v, av, bv, cv, mv, pv,
                   sem0, sem1, sem2, sem3):
    cid = lax.axis_index("c")
    sid = lax.axis_index("s")
    wid = sid * _NC + cid
    sems = (sem0, sem1, sem2, sem3)

    idx_copies = [
        pltpu.async_copy(
            idx_hbm.at[pl.ds(wid * _B_PER_W + j * _CHUNK, _CHUNK)],
            idx_v.at[pl.ds(j * _CHUNK, _CHUNK)], sems[j])
        for j in range(_NCHUNK)
    ]

    copies = []
    for j in range(_NCHUNK):
        s = pl.ds(j * _CHUNK, _CHUNK)
        idx_copies[j].wait()
        copies.append((pltpu.async_copy(a_hbm.at[idx_v.at[s]], av.at[s], sems[j]),
                       pltpu.async_copy(b_hbm.at[idx_v.at[s]], bv.at[s], sems[j]),
                       pltpu.async_copy(c_hbm.at[idx_v.at[s]], cv.at[s], sems[j])))
    pltpu.sync_copy(m_hbm, mv.at[pl.ds(0, 1)])

    acc_ac = jnp.zeros((_L,), jnp.float32)
    acc_b = jnp.zeros((_L,), jnp.float32)
    for j in range(_NCHUNK):
        for cp in copies[j]:
            cp.wait()

        def body(i, accs):
            a_ac, a_b = accs
            s = pl.ds(j * _CHUNK + i * _L, _L)
            return a_ac + av[s] + cv[s], a_b + bv[s]

        acc_ac, acc_b = lax.fori_loop(0, _CHUNK // _L, body,
                                      (acc_ac, acc_b), unroll=2)

    m = mv[...][0]
    pv[...] = acc_ac * m + acc_b
    pltpu.sync_copy(pv, out_hbm.at[wid])


def kernel(c_t_indices, M_prev, A, B, C):
    m1 = jnp.reshape(M_prev, (1,)).astype(jnp.float32)
    partials = _sc_gather_sum(c_t_indices.astype(jnp.int32), A, B, C, m1)
    return jnp.sum(partials)


# speedup vs baseline: 1.0348x; 1.0004x over previous
import functools

import jax
import jax.numpy as jnp
from jax import lax
from jax.experimental import pallas as pl
from jax.experimental.pallas import tpu as pltpu
from jax.experimental.pallas import tpu_sc as plsc

_BATCH = 16384
_L = 16
_NC = 2
_NS = 16
_NW = _NC * _NS
_B_PER_W = _BATCH // _NW
_CHUNK = 128
_NCHUNK = _B_PER_W // _CHUNK

_mesh = plsc.VectorSubcoreMesh(core_axis_name="c", subcore_axis_name="s")


@functools.partial(
    pl.kernel,
    mesh=_mesh,
    out_type=jax.ShapeDtypeStruct((_NW, _L), jnp.float32),
    scratch_types=[
        pltpu.VMEM((_B_PER_W,), jnp.int32),
        pltpu.VMEM((_B_PER_W,), jnp.float32),
        pltpu.VMEM((_B_PER_W,), jnp.float32),
        pltpu.VMEM((_B_PER_W,), jnp.float32),
        pltpu.VMEM((_L,), jnp.float32),
        pltpu.VMEM((_L,), jnp.float32),
        pltpu.SemaphoreType.DMA,
        pltpu.SemaphoreType.DMA,
        pltpu.SemaphoreType.DMA,
        pltpu.SemaphoreType.DMA,
        pltpu.SemaphoreType.DMA,
    ],
)
def _sc_gather_sum(idx_hbm, a_hbm, b_hbm, c_hbm, m_hbm, out_hbm,
                   idx_v, av, bv, cv, mv, pv,
                   sem0, sem1, sem2, sem3, sem4):
    cid = lax.axis_index("c")
    sid = lax.axis_index("s")
    wid = sid * _NC + cid
    sems = (sem0, sem1, sem2, sem3)

    m_copy = pltpu.async_copy(m_hbm, mv.at[pl.ds(0, 1)], sem4)

    idx_copies = [
        pltpu.async_copy(
            idx_hbm.at[pl.ds(wid * _B_PER_W + j * _CHUNK, _CHUNK)],
            idx_v.at[pl.ds(j * _CHUNK, _CHUNK)], sems[j])
        for j in range(_NCHUNK)
    ]

    copies = []
    for j in range(_NCHUNK):
        s = pl.ds(j * _CHUNK, _CHUNK)
        idx_copies[j].wait()
        copies.append((pltpu.async_copy(a_hbm.at[idx_v.at[s]], av.at[s], sems[j]),
                       pltpu.async_copy(b_hbm.at[idx_v.at[s]], bv.at[s], sems[j]),
                       pltpu.async_copy(c_hbm.at[idx_v.at[s]], cv.at[s], sems[j])))
    acc_ac = jnp.zeros((_L,), jnp.float32)
    acc_b = jnp.zeros((_L,), jnp.float32)
    for j in range(_NCHUNK):
        for cp in copies[j]:
            cp.wait()

        def body(i, accs):
            a_ac, a_b = accs
            s = pl.ds(j * _CHUNK + i * _L, _L)
            return a_ac + av[s] + cv[s], a_b + bv[s]

        acc_ac, acc_b = lax.fori_loop(0, _CHUNK // _L, body,
                                      (acc_ac, acc_b), unroll=2)

    m_copy.wait()
    m = mv[...][0]
    pv[...] = acc_ac * m + acc_b
    pltpu.sync_copy(pv, out_hbm.at[wid])


def kernel(c_t_indices, M_prev, A, B, C):
    m1 = jnp.reshape(M_prev, (1,)).astype(jnp.float32)
    partials = _sc_gather_sum(c_t_indices.astype(jnp.int32), A, B, C, m1)
    return jnp.sum(partials)
